# baseline (speedup 1.0000x reference)
import functools

import jax
import jax.numpy as jnp
from jax import lax
from jax.experimental import pallas as pl
from jax.experimental.pallas import tpu as pltpu
from jax.experimental.pallas import tpu_sc as plsc

N = 10000
E = 320000
H = 128
EPS_BN = 1e-5
EPS_NORM = 1e-6

NC = 2
NS = 16
NW = NC * NS
EW = E // NW
B = 80
NCHUNK = EW // B
N2 = 10240
RT = N2 // NS
ZB = 128

_mesh = plsc.VectorSubcoreMesh(core_axis_name="c", subcore_axis_name="s")


def _mm_body(x_ref, w_ref, b_ref, o_ref):
    o_ref[...] = lax.dot_general(
        x_ref[...], w_ref[...], (((1,), (0,)), ((), ())),
        preferred_element_type=jnp.float32) + b_ref[...]


def _matmul_bias(x, w, b, block_rows):
    m, k = x.shape
    kw, n = w.shape
    grid = m // block_rows
    return pl.pallas_call(
        _mm_body,
        grid=(grid,),
        in_specs=[
            pl.BlockSpec((block_rows, k), lambda i: (i, 0)),
            pl.BlockSpec((kw, n), lambda i: (0, 0)),
            pl.BlockSpec((1, n), lambda i: (0, 0)),
        ],
        out_specs=pl.BlockSpec((block_rows, n), lambda i: (i, 0)),
        out_shape=jax.ShapeDtypeStruct((m, n), jnp.float32),
    )(x, w, b.reshape(1, n))


@functools.partial(
    pl.kernel,
    out_type=(jax.ShapeDtypeStruct((E, H), jnp.float32),
              jax.ShapeDtypeStruct((NW, 16, 16), jnp.float32)),
    mesh=_mesh,
    scratch_types=(
        pltpu.VMEM((B,), jnp.int32),
        pltpu.VMEM((B,), jnp.int32),
        pltpu.VMEM((B, H), jnp.float32),
        pltpu.VMEM((B, H), jnp.float32),
        pltpu.VMEM((B, H), jnp.float32),
        pltpu.VMEM((16, 16), jnp.float32),
        pltpu.SemaphoreType.DMA,
        pltpu.SemaphoreType.DMA,
    ),
)
def _s1(src_hbm, dst_hbm, ps_hbm, pd_hbm, m_hbm, z_hbm, stats_hbm,
        si_v, di_v, m_v, gs_v, gd_v, acc_v, sem1, sem2):
    c = lax.axis_index("c")
    s = lax.axis_index("s")
    wid = c * NS + s
    base0 = wid * EW
    zero16 = jnp.zeros((16,), jnp.float32)
    for j in range(16):
        acc_v[j] = zero16

    def chunk_body(i, carry):
        base = base0 + i * B
        pltpu.sync_copy(src_hbm.at[pl.ds(base, B)], si_v)
        pltpu.sync_copy(dst_hbm.at[pl.ds(base, B)], di_v)
        cp_m = pltpu.async_copy(m_hbm.at[pl.ds(base, B), :], m_v, sem1)
        cp_s = pltpu.async_copy(ps_hbm.at[si_v], gs_v, sem2)
        cp_d = pltpu.async_copy(pd_hbm.at[di_v], gd_v, sem2)
        cp_m.wait()
        cp_s.wait()
        cp_d.wait()

        def row_body(r, rc):
            for j in range(8):
                sl = pl.ds(j * 16, 16)
                z = m_v[r, sl] + gs_v[r, sl] + gd_v[r, sl]
                m_v[r, sl] = z
                acc_v[j] = acc_v[j] + z
                acc_v[8 + j] = acc_v[8 + j] + z * z
            return rc

        lax.fori_loop(0, B, row_body, 0)
        pltpu.sync_copy(m_v, z_hbm.at[pl.ds(base, B), :])
        return carry

    lax.fori_loop(0, NCHUNK, chunk_body, 0)
    pltpu.sync_copy(acc_v, stats_hbm.at[wid])


def _bn1_body(stats_ref, g_ref, b_ref, o_ref):
    st = jnp.sum(stats_ref[...], axis=0)
    mean = st[0:8] / float(E)
    var = st[8:16] / float(E) - mean * mean
    scale = g_ref[...] * lax.rsqrt(var + EPS_BN)
    shift = b_ref[...] - mean * scale
    o_ref[0:8] = scale
    o_ref[8:16] = shift


def _bn1(stats, g, b):
    out = pl.pallas_call(
        _bn1_body,
        out_shape=jax.ShapeDtypeStruct((16, 16), jnp.float32),
    )(stats, g.reshape(8, 16), b.reshape(8, 16))
    return out.reshape(2, H)


@functools.partial(
    pl.kernel,
    out_type=(jax.ShapeDtypeStruct((E, H), jnp.float32),
              jax.ShapeDtypeStruct((NC, N2, H), jnp.float32)),
    mesh=_mesh,
    scratch_types=(
        pltpu.VMEM((B,), jnp.int32),
        pltpu.VMEM((B, H), jnp.float32),
        pltpu.VMEM((B, H), jnp.float32),
        pltpu.VMEM((2, H), jnp.float32),
        pltpu.VMEM((ZB, H), jnp.float32),
        pltpu.VMEM_SHARED((N2, H), jnp.float32),
        pltpu.SemaphoreType.DMA,
    ),
)
def _s2(src_hbm, z_hbm, coef_hbm, uef_hbm, aggp_hbm,
        si_v, z_v, sig_v, coef_v, zrow_v, shared, sem1):
    c = lax.axis_index("c")
    s = lax.axis_index("s")
    wid = c * NS + s
    base0 = wid * EW
    zero16 = jnp.zeros((16,), jnp.float32)

    def zr(r, carry):
        for j in range(8):
            zrow_v[r, pl.ds(j * 16, 16)] = zero16
        return carry

    lax.fori_loop(0, ZB, zr, 0)
    for k in range(5):
        pltpu.sync_copy(zrow_v, shared.at[pl.ds(s * RT + k * ZB, ZB), :])
    pltpu.sync_copy(coef_hbm, coef_v)
    plsc.subcore_barrier()

    def chunk_body(i, carry):
        base = base0 + i * B
        pltpu.sync_copy(src_hbm.at[pl.ds(base, B)], si_v)
        pltpu.sync_copy(z_hbm.at[pl.ds(base, B), :], z_v)

        def row_body(r, rc):
            for j in range(8):
                sl = pl.ds(j * 16, 16)
                y = z_v[r, sl] * coef_v[0, sl] + coef_v[1, sl]
                sg = 1.0 / (1.0 + jnp.exp(-y))
                u = y * sg
                z_v[r, sl] = u
                sig_v[r, sl] = 1.0 / (1.0 + jnp.exp(-u))
            return rc

        lax.fori_loop(0, B, row_body, 0)
        pltpu.sync_copy(z_v, uef_hbm.at[pl.ds(base, B), :])
        pltpu.sync_copy(sig_v, shared.at[si_v], add=True)
        return carry

    lax.fori_loop(0, NCHUNK, chunk_body, 0)
    plsc.subcore_barrier()
    for k in range(5):
        sl = pl.ds(s * RT + k * ZB, ZB)
        pltpu.sync_copy(shared.at[sl, :], aggp_hbm.at[c, sl, :])


@functools.partial(
    pl.kernel,
    out_type=jax.ShapeDtypeStruct((NC, N2, H), jnp.float32),
    mesh=_mesh,
    scratch_types=(
        pltpu.VMEM((B,), jnp.int32),
        pltpu.VMEM((B,), jnp.int32),
        pltpu.VMEM((B, H), jnp.float32),
        pltpu.VMEM((B, H), jnp.float32),
        pltpu.VMEM((ZB, H), jnp.float32),
        pltpu.VMEM_SHARED((N2, H), jnp.float32),
        pltpu.SemaphoreType.DMA,
        pltpu.SemaphoreType.DMA,
    ),
)
def _s3(src_hbm, dst_hbm, uef_hbm, q_hbm, sp_hbm,
        si_v, di_v, u_v, q_v, zrow_v, shared, sem1, sem2):
    c = lax.axis_index("c")
    s = lax.axis_index("s")
    wid = c * NS + s
    base0 = wid * EW
    zero16 = jnp.zeros((16,), jnp.float32)

    def zr(r, carry):
        for j in range(8):
            zrow_v[r, pl.ds(j * 16, 16)] = zero16
        return carry

    lax.fori_loop(0, ZB, zr, 0)
    for k in range(5):
        pltpu.sync_copy(zrow_v, shared.at[pl.ds(s * RT + k * ZB, ZB), :])
    plsc.subcore_barrier()

    def chunk_body(i, carry):
        base = base0 + i * B
        pltpu.sync_copy(src_hbm.at[pl.ds(base, B)], si_v)
        pltpu.sync_copy(dst_hbm.at[pl.ds(base, B)], di_v)
        cp_u = pltpu.async_copy(uef_hbm.at[pl.ds(base, B), :], u_v, sem1)
        cp_q = pltpu.async_copy(q_hbm.at[di_v], q_v, sem2)
        cp_u.wait()
        cp_q.wait()

        def row_body(r, rc):
            for j in range(8):
                sl = pl.ds(j * 16, 16)
                u = u_v[r, sl]
                sg = 1.0 / (1.0 + jnp.exp(-u))
                u_v[r, sl] = sg * q_v[r, sl]
            return rc

        lax.fori_loop(0, B, row_body, 0)
        pltpu.sync_copy(u_v, shared.at[si_v], add=True)
        return carry

    lax.fori_loop(0, NCHUNK, chunk_body, 0)
    plsc.subcore_barrier()
    for k in range(5):
        sl = pl.ds(s * RT + k * ZB, ZB)
        pltpu.sync_copy(shared.at[sl, :], sp_hbm.at[c, sl, :])


def _post_body(nf_ref, r_ref, aggp_ref, sp_ref, g_ref, b_ref, o_ref):
    agg = aggp_ref[0, :N] + aggp_ref[1, :N] + EPS_NORM
    sden = sp_ref[0, :N] + sp_ref[1, :N]
    an = r_ref[...] + sden / agg
    mean = jnp.mean(an, axis=0, keepdims=True)
    var = jnp.mean(an * an, axis=0, keepdims=True) - mean * mean
    y = (an - mean) * lax.rsqrt(var + EPS_BN) * g_ref[...] + b_ref[...]
    o_ref[...] = nf_ref[...] + y * (1.0 / (1.0 + jnp.exp(-y)))


def _post(nf, r, aggp, sp, g, b):
    return pl.pallas_call(
        _post_body,
        out_shape=jax.ShapeDtypeStruct((N, H), jnp.float32),
    )(nf, r, aggp, sp, g.reshape(1, H), b.reshape(1, H))


def kernel(edge_index, node_features, edge_features, W_gate, b_gate,
           W_src, b_src, W_dst, b_dst, g_z, b_z, g_node, b_node):
    src = edge_index[0]
    dst = edge_index[1]

    w_edge = W_gate[:, 2 * H:].T
    w_node = jnp.concatenate(
        [W_gate[:, :H].T, W_gate[:, H:2 * H].T, W_dst.T, W_src.T], axis=1)
    b_node_cat = jnp.concatenate(
        [jnp.zeros((2 * H,), jnp.float32), b_dst, b_src])

    m = _matmul_bias(edge_features, w_edge, b_gate, 2000)
    pcat = _matmul_bias(node_features, w_node, b_node_cat, 1000)
    ps = pcat[:, :H]
    pd = pcat[:, H:2 * H]
    q = pcat[:, 2 * H:3 * H]
    r = pcat[:, 3 * H:]

    z, stats = _s1(src, dst, ps, pd, m)
    coef = _bn1(stats, g_z, b_z)
    uef, aggp = _s2(src, z, coef)
    sp = _s3(src, dst, uef, q)
    unf = _post(node_features, r, aggp, sp, g_node, b_node)
    return (unf, uef)
